# baseline (speedup 1.0000x reference)
import functools

import jax
import jax.numpy as jnp
from jax import lax
from jax.experimental import pallas as pl
from jax.experimental.pallas import tpu as pltpu
from jax.experimental.pallas import tpu_sc as plsc

B = 16384
H = 64
NC = 2
NS = 16
NW = NC * NS
B_PER_W = B // NW

TC_BLOCK = 1024
GRID = B // TC_BLOCK

STRIPE = 2048
N_USER = 190662
N_MUSIC = 42800
N_SINGER = 417


def _packed_rows(m):
    return ((m + STRIPE - 1) // STRIPE) * (STRIPE // 2)


def _pack_kernel(tab_t_ref, eye_ref, out_ref):
    dd = (((0,), (0,)), ((), ()))
    left = lax.dot_general(tab_t_ref[:, 0:STRIPE // 2], eye_ref[...], dd)
    right = lax.dot_general(tab_t_ref[:, STRIPE // 2:STRIPE], eye_ref[...], dd)
    out_ref[...] = jnp.concatenate([left, right], axis=1)


def _pack_table(table_t, m, eye):
    grid = (m + STRIPE - 1) // STRIPE
    return pl.pallas_call(
        _pack_kernel,
        grid=(grid,),
        in_specs=[pl.BlockSpec((H, STRIPE), lambda i: (0, i)),
                  pl.BlockSpec((H, H), lambda i: (0, 0))],
        out_specs=pl.BlockSpec((STRIPE // 2, 2 * H), lambda i: (i, 0)),
        out_shape=jax.ShapeDtypeStruct((grid * (STRIPE // 2), 2 * H),
                                       jnp.float32),
    )(table_t, eye)


def _sc_gather_kernel(user_p, singer_p, music_p, idx0, idx1, idx2,
                      out0, out1, out2, idx_v, rows_v, sem):
    wid = lax.axis_index("s") * NC + lax.axis_index("c")
    base = wid * B_PER_W
    rows = pl.ds(base, B_PER_W)
    work = ((user_p, idx0, out0), (singer_p, idx1, out1),
            (music_p, idx2, out2))
    for table, idx_hbm, out_hbm in work:
        pltpu.sync_copy(idx_hbm.at[rows], idx_v)
        pltpu.async_copy(table.at[idx_v], rows_v, sem).wait()
        pltpu.sync_copy(rows_v, out_hbm.at[rows])


def _sc_gather(user_p, singer_p, music_p, idx0, idx1, idx2):
    mesh = plsc.VectorSubcoreMesh(core_axis_name="c", subcore_axis_name="s")
    out = jax.ShapeDtypeStruct((B, 2 * H), jnp.float32)
    k = pl.kernel(
        _sc_gather_kernel,
        out_type=(out, out, out),
        mesh=mesh,
        compiler_params=pltpu.CompilerParams(use_tc_tiling_on_sc=True),
        scratch_types=[
            pltpu.VMEM((B_PER_W,), jnp.int32),
            pltpu.VMEM((B_PER_W, 2 * H), jnp.float32),
            pltpu.SemaphoreType.DMA,
        ],
    )
    return k(user_p, singer_p, music_p, idx0, idx1, idx2)


def _tc_kernel(gath0_ref, gath1_ref, gath2_ref, art_t_ref, mom_t_ref,
               feat_ref, ids_ref, wuf_ref, wml_ref, wsf_ref, bias_t_ref,
               age_emb_ref, gender_emb_ref, genre_emb_ref, out_ref):
    f32 = jnp.float32
    hi = jax.lax.Precision.DEFAULT

    def onehot_lookup_t(f, n, table_ref):
        ids_row = ids_ref[f]
        iota = lax.broadcasted_iota(jnp.int32, (n, TC_BLOCK), 0)
        oh = (iota == ids_row).astype(f32)
        return lax.dot_general(table_ref[...], oh,
                               (((0,), (0,)), ((), ())), precision=hi)

    def half_select_t(g_ref, par_row):
        g_t = g_ref[...].T
        return jnp.where(par_row == 1, g_t[H:2 * H, :], g_t[0:H, :])

    age_t = onehot_lookup_t(0, 6, age_emb_ref)
    gender_t = onehot_lookup_t(1, 2, gender_emb_ref)
    genre_t = onehot_lookup_t(2, 18, genre_emb_ref)

    uemb_t = half_select_t(gath0_ref, ids_ref[3])
    semb_t = half_select_t(gath1_ref, ids_ref[4])
    memb_t = half_select_t(gath2_ref, ids_ref[5])

    art_t = lax.dot_general(wuf_ref[...], art_t_ref[...],
                            (((1,), (0,)), ((), ())),
                            precision=hi) + bias_t_ref[:, 0:1]
    mom_t = lax.dot_general(wml_ref[...], mom_t_ref[...],
                            (((1,), (0,)), ((), ())),
                            precision=hi) + bias_t_ref[:, 1:2]
    feat_t = lax.dot_general(wsf_ref[...], feat_ref[...],
                             (((1,), (1,)), ((), ())),
                             precision=hi) + bias_t_ref[:, 2:3]

    out_ref[...] = jnp.concatenate(
        [uemb_t, age_t, gender_t, art_t, mom_t, feat_t,
         semb_t, genre_t, memb_t], axis=0)


def _tc_assemble(gath0, gath1, gath2, art_t, mom_t, features, ids,
                 wuf, wml, wsf, biases_t, age_emb, gender_emb, genre_emb):
    return pl.pallas_call(
        _tc_kernel,
        grid=(GRID,),
        in_specs=[
            pl.BlockSpec((TC_BLOCK, 2 * H), lambda i: (i, 0)),
            pl.BlockSpec((TC_BLOCK, 2 * H), lambda i: (i, 0)),
            pl.BlockSpec((TC_BLOCK, 2 * H), lambda i: (i, 0)),
            pl.BlockSpec((200, TC_BLOCK), lambda i: (0, i)),
            pl.BlockSpec((64, TC_BLOCK), lambda i: (0, i)),
            pl.BlockSpec((TC_BLOCK, 128), lambda i: (i, 0)),
            pl.BlockSpec((6, 1, TC_BLOCK), lambda i: (0, 0, i)),
            pl.BlockSpec((64, 200), lambda i: (0, 0)),
            pl.BlockSpec((64, 64), lambda i: (0, 0)),
            pl.BlockSpec((64, 128), lambda i: (0, 0)),
            pl.BlockSpec((64, 3), lambda i: (0, 0)),
            pl.BlockSpec((6, H), lambda i: (0, 0)),
            pl.BlockSpec((2, H), lambda i: (0, 0)),
            pl.BlockSpec((18, H), lambda i: (0, 0)),
        ],
        out_specs=pl.BlockSpec((9 * H, TC_BLOCK), lambda i: (0, i)),
        out_shape=jax.ShapeDtypeStruct((9 * H, B), jnp.float32),
    )(gath0, gath1, gath2, art_t, mom_t, features, ids,
      wuf, wml, wsf, biases_t, age_emb, gender_emb, genre_emb)


def kernel(user_articles, user_moments, user_id, user_age, user_gender,
           music_features, music_singer, music_genre, music_id,
           W_uf, b_uf, W_ml, b_ml, W_sf, b_sf,
           UserEmb, AgeEmb, GenderEmb, SingerEmb, GenreEmb, MusicEmb):
    i32 = jnp.int32

    eye = jnp.eye(H, dtype=jnp.float32)
    user_p = _pack_table(UserEmb.T, N_USER, eye)
    singer_p = _pack_table(SingerEmb.T, N_SINGER, eye)
    music_p = _pack_table(MusicEmb.T, N_MUSIC, eye)

    def packed_row_parity(r):
        r = r.astype(i32)
        row = (r // STRIPE) * (STRIPE // 2) + (r % (STRIPE // 2))
        parity = (r // (STRIPE // 2)) % 2
        return row, parity

    row_u, par_u = packed_row_parity(user_id)
    row_s, par_s = packed_row_parity(music_singer.reshape(B))
    row_m, par_m = packed_row_parity(music_id.reshape(B))

    gath0, gath1, gath2 = _sc_gather(user_p, singer_p, music_p,
                                     row_u, row_s, row_m)

    ids = jnp.stack([user_age.astype(i32),
                     user_gender.astype(i32),
                     music_genre.reshape(B).astype(i32),
                     par_u, par_s, par_m], axis=0).reshape(6, 1, B)
    biases_t = jnp.stack([b_uf, b_ml, b_sf], axis=1)
    out_t = _tc_assemble(gath0, gath1, gath2, user_articles.T,
                         user_moments.T, music_features.reshape(B, 128),
                         ids, W_uf, W_ml, W_sf, biases_t,
                         AgeEmb, GenderEmb, GenreEmb)
    return out_t.T


# speedup vs baseline: 1.3149x; 1.3149x over previous
import functools

import jax
import jax.numpy as jnp
from jax import lax
from jax.experimental import pallas as pl
from jax.experimental.pallas import tpu as pltpu
from jax.experimental.pallas import tpu_sc as plsc

B = 16384
H = 64
NC = 2
NS = 16
NW = NC * NS
B_PER_W = B // NW

TC_BLOCK = 1024
GRID = B // TC_BLOCK

STRIPE = 8192
N_USER = 190662
N_MUSIC = 42800
N_SINGER = 417


def _packed_rows(m):
    return ((m + STRIPE - 1) // STRIPE) * (STRIPE // 2)


def _pack_kernel(tab_t_ref, out_ref):
    left = tab_t_ref[:, 0:STRIPE // 2].T
    right = tab_t_ref[:, STRIPE // 2:STRIPE].T
    out_ref[...] = jnp.concatenate([left, right], axis=1)


def _pack_table(table_t, m):
    grid = (m + STRIPE - 1) // STRIPE
    return pl.pallas_call(
        _pack_kernel,
        grid=(grid,),
        in_specs=[pl.BlockSpec((H, STRIPE), lambda i: (0, i))],
        out_specs=pl.BlockSpec((STRIPE // 2, 2 * H), lambda i: (i, 0)),
        out_shape=jax.ShapeDtypeStruct((grid * (STRIPE // 2), 2 * H),
                                       jnp.float32),
    )(table_t)


def _sc_gather_kernel(user_p, singer_p, music_p, idx0, idx1, idx2,
                      out0, out1, out2, idx_v, rows_v, sem):
    wid = lax.axis_index("s") * NC + lax.axis_index("c")
    base = wid * B_PER_W
    rows = pl.ds(base, B_PER_W)
    work = ((user_p, idx0, out0), (singer_p, idx1, out1),
            (music_p, idx2, out2))
    for table, idx_hbm, out_hbm in work:
        pltpu.sync_copy(idx_hbm.at[rows], idx_v)
        pltpu.async_copy(table.at[idx_v], rows_v, sem).wait()
        pltpu.sync_copy(rows_v, out_hbm.at[rows])


def _sc_gather(user_p, singer_p, music_p, idx0, idx1, idx2):
    mesh = plsc.VectorSubcoreMesh(core_axis_name="c", subcore_axis_name="s")
    out = jax.ShapeDtypeStruct((B, 2 * H), jnp.float32)
    k = pl.kernel(
        _sc_gather_kernel,
        out_type=(out, out, out),
        mesh=mesh,
        compiler_params=pltpu.CompilerParams(use_tc_tiling_on_sc=True),
        scratch_types=[
            pltpu.VMEM((B_PER_W,), jnp.int32),
            pltpu.VMEM((B_PER_W, 2 * H), jnp.float32),
            pltpu.SemaphoreType.DMA,
        ],
    )
    return k(user_p, singer_p, music_p, idx0, idx1, idx2)


def _tc_kernel(gath0_ref, gath1_ref, gath2_ref, art_t_ref, mom_t_ref,
               feat_ref, ids_ref, wuf_ref, wml_ref, wsf_ref, bias_t_ref,
               age_emb_ref, gender_emb_ref, genre_emb_ref, out_ref):
    f32 = jnp.float32
    hi = jax.lax.Precision.DEFAULT

    def onehot_lookup_t(f, n, table_ref):
        ids_row = ids_ref[f]
        iota = lax.broadcasted_iota(jnp.int32, (n, TC_BLOCK), 0)
        oh = (iota == ids_row).astype(f32)
        return lax.dot_general(table_ref[...], oh,
                               (((0,), (0,)), ((), ())), precision=hi)

    def half_select_t(g_ref, par_row):
        g_t = g_ref[...].T
        return jnp.where(par_row == 1, g_t[H:2 * H, :], g_t[0:H, :])

    age_t = onehot_lookup_t(0, 6, age_emb_ref)
    gender_t = onehot_lookup_t(1, 2, gender_emb_ref)
    genre_t = onehot_lookup_t(2, 18, genre_emb_ref)

    uemb_t = half_select_t(gath0_ref, ids_ref[3])
    semb_t = half_select_t(gath1_ref, ids_ref[4])
    memb_t = half_select_t(gath2_ref, ids_ref[5])

    art_t = lax.dot_general(wuf_ref[...], art_t_ref[...],
                            (((1,), (0,)), ((), ())),
                            precision=hi) + bias_t_ref[:, 0:1]
    mom_t = lax.dot_general(wml_ref[...], mom_t_ref[...],
                            (((1,), (0,)), ((), ())),
                            precision=hi) + bias_t_ref[:, 1:2]
    feat_t = lax.dot_general(wsf_ref[...], feat_ref[...],
                             (((1,), (1,)), ((), ())),
                             precision=hi) + bias_t_ref[:, 2:3]

    out_ref[...] = jnp.concatenate(
        [uemb_t, age_t, gender_t, art_t, mom_t, feat_t,
         semb_t, genre_t, memb_t], axis=0)


def _tc_assemble(gath0, gath1, gath2, art_t, mom_t, features, ids,
                 wuf, wml, wsf, biases_t, age_emb, gender_emb, genre_emb):
    return pl.pallas_call(
        _tc_kernel,
        grid=(GRID,),
        in_specs=[
            pl.BlockSpec((TC_BLOCK, 2 * H), lambda i: (i, 0)),
            pl.BlockSpec((TC_BLOCK, 2 * H), lambda i: (i, 0)),
            pl.BlockSpec((TC_BLOCK, 2 * H), lambda i: (i, 0)),
            pl.BlockSpec((200, TC_BLOCK), lambda i: (0, i)),
            pl.BlockSpec((64, TC_BLOCK), lambda i: (0, i)),
            pl.BlockSpec((TC_BLOCK, 128), lambda i: (i, 0)),
            pl.BlockSpec((6, 1, TC_BLOCK), lambda i: (0, 0, i)),
            pl.BlockSpec((64, 200), lambda i: (0, 0)),
            pl.BlockSpec((64, 64), lambda i: (0, 0)),
            pl.BlockSpec((64, 128), lambda i: (0, 0)),
            pl.BlockSpec((64, 3), lambda i: (0, 0)),
            pl.BlockSpec((6, H), lambda i: (0, 0)),
            pl.BlockSpec((2, H), lambda i: (0, 0)),
            pl.BlockSpec((18, H), lambda i: (0, 0)),
        ],
        out_specs=pl.BlockSpec((9 * H, TC_BLOCK), lambda i: (0, i)),
        out_shape=jax.ShapeDtypeStruct((9 * H, B), jnp.float32),
    )(gath0, gath1, gath2, art_t, mom_t, features, ids,
      wuf, wml, wsf, biases_t, age_emb, gender_emb, genre_emb)


def kernel(user_articles, user_moments, user_id, user_age, user_gender,
           music_features, music_singer, music_genre, music_id,
           W_uf, b_uf, W_ml, b_ml, W_sf, b_sf,
           UserEmb, AgeEmb, GenderEmb, SingerEmb, GenreEmb, MusicEmb):
    i32 = jnp.int32

    user_p = _pack_table(UserEmb.T, N_USER)
    singer_p = _pack_table(SingerEmb.T, N_SINGER)
    music_p = _pack_table(MusicEmb.T, N_MUSIC)

    def packed_row_parity(r):
        r = r.astype(i32)
        row = (r // STRIPE) * (STRIPE // 2) + (r % (STRIPE // 2))
        parity = (r // (STRIPE // 2)) % 2
        return row, parity

    row_u, par_u = packed_row_parity(user_id)
    row_s, par_s = packed_row_parity(music_singer.reshape(B))
    row_m, par_m = packed_row_parity(music_id.reshape(B))

    gath0, gath1, gath2 = _sc_gather(user_p, singer_p, music_p,
                                     row_u, row_s, row_m)

    ids = jnp.stack([user_age.astype(i32),
                     user_gender.astype(i32),
                     music_genre.reshape(B).astype(i32),
                     par_u, par_s, par_m], axis=0).reshape(6, 1, B)
    biases_t = jnp.stack([b_uf, b_ml, b_sf], axis=1)
    out_t = _tc_assemble(gath0, gath1, gath2, user_articles.T,
                         user_moments.T, music_features.reshape(B, 128),
                         ids, W_uf, W_ml, W_sf, biases_t,
                         AgeEmb, GenderEmb, GenreEmb)
    return out_t.T


# speedup vs baseline: 1.3165x; 1.0012x over previous
import functools

import jax
import jax.numpy as jnp
from jax import lax
from jax.experimental import pallas as pl
from jax.experimental.pallas import tpu as pltpu
from jax.experimental.pallas import tpu_sc as plsc

B = 16384
H = 64
NC = 2
NS = 16
NW = NC * NS
B_PER_W = B // NW

TC_BLOCK = 1024
GRID = B // TC_BLOCK

STRIPE = 8192
N_USER = 190662
N_MUSIC = 42800
N_SINGER = 417


def _packed_rows(m):
    return ((m + STRIPE - 1) // STRIPE) * (STRIPE // 2)


def _pack_kernel(tab_t_ref, out_ref):
    left = tab_t_ref[:, 0:STRIPE // 2].T
    right = tab_t_ref[:, STRIPE // 2:STRIPE].T
    out_ref[...] = jnp.concatenate([left, right], axis=1)


def _pack_table(table_t, m):
    grid = (m + STRIPE - 1) // STRIPE
    return pl.pallas_call(
        _pack_kernel,
        grid=(grid,),
        compiler_params=pltpu.CompilerParams(
            dimension_semantics=("parallel",)),
        in_specs=[pl.BlockSpec((H, STRIPE), lambda i: (0, i))],
        out_specs=pl.BlockSpec((STRIPE // 2, 2 * H), lambda i: (i, 0)),
        out_shape=jax.ShapeDtypeStruct((grid * (STRIPE // 2), 2 * H),
                                       jnp.float32),
    )(table_t)


def _sc_gather_kernel(user_p, singer_p, music_p, idx0, idx1, idx2,
                      out0, out1, out2, idx_v, rows_v, sem):
    wid = lax.axis_index("s") * NC + lax.axis_index("c")
    base = wid * B_PER_W
    rows = pl.ds(base, B_PER_W)
    work = ((user_p, idx0, out0), (singer_p, idx1, out1),
            (music_p, idx2, out2))
    for table, idx_hbm, out_hbm in work:
        pltpu.sync_copy(idx_hbm.at[rows], idx_v)
        pltpu.async_copy(table.at[idx_v], rows_v, sem).wait()
        pltpu.sync_copy(rows_v, out_hbm.at[rows])


def _sc_gather(user_p, singer_p, music_p, idx0, idx1, idx2):
    mesh = plsc.VectorSubcoreMesh(core_axis_name="c", subcore_axis_name="s")
    out = jax.ShapeDtypeStruct((B, 2 * H), jnp.float32)
    k = pl.kernel(
        _sc_gather_kernel,
        out_type=(out, out, out),
        mesh=mesh,
        compiler_params=pltpu.CompilerParams(use_tc_tiling_on_sc=True),
        scratch_types=[
            pltpu.VMEM((B_PER_W,), jnp.int32),
            pltpu.VMEM((B_PER_W, 2 * H), jnp.float32),
            pltpu.SemaphoreType.DMA,
        ],
    )
    return k(user_p, singer_p, music_p, idx0, idx1, idx2)


def _tc_kernel(gath0_ref, gath1_ref, gath2_ref, art_t_ref, mom_t_ref,
               feat_ref, ids_ref, wuf_ref, wml_ref, wsf_ref, bias_t_ref,
               age_emb_ref, gender_emb_ref, genre_emb_ref, out_ref):
    f32 = jnp.float32
    hi = jax.lax.Precision.DEFAULT

    def onehot_lookup_t(f, n, table_ref):
        ids_row = ids_ref[f]
        iota = lax.broadcasted_iota(jnp.int32, (n, TC_BLOCK), 0)
        oh = (iota == ids_row).astype(f32)
        return lax.dot_general(table_ref[...], oh,
                               (((0,), (0,)), ((), ())), precision=hi)

    def half_select_t(g_ref, par_row):
        g_t = g_ref[...].T
        return jnp.where(par_row == 1, g_t[H:2 * H, :], g_t[0:H, :])

    age_t = onehot_lookup_t(0, 6, age_emb_ref)
    gender_t = onehot_lookup_t(1, 2, gender_emb_ref)
    genre_t = onehot_lookup_t(2, 18, genre_emb_ref)

    uemb_t = half_select_t(gath0_ref, ids_ref[3])
    semb_t = half_select_t(gath1_ref, ids_ref[4])
    memb_t = half_select_t(gath2_ref, ids_ref[5])

    art_t = lax.dot_general(wuf_ref[...], art_t_ref[...],
                            (((1,), (0,)), ((), ())),
                            precision=hi) + bias_t_ref[:, 0:1]
    mom_t = lax.dot_general(wml_ref[...], mom_t_ref[...],
                            (((1,), (0,)), ((), ())),
                            precision=hi) + bias_t_ref[:, 1:2]
    feat_t = lax.dot_general(wsf_ref[...], feat_ref[...],
                             (((1,), (1,)), ((), ())),
                             precision=hi) + bias_t_ref[:, 2:3]

    out_ref[...] = jnp.concatenate(
        [uemb_t, age_t, gender_t, art_t, mom_t, feat_t,
         semb_t, genre_t, memb_t], axis=0)


def _tc_assemble(gath0, gath1, gath2, art_t, mom_t, features, ids,
                 wuf, wml, wsf, biases_t, age_emb, gender_emb, genre_emb):
    return pl.pallas_call(
        _tc_kernel,
        grid=(GRID,),
        compiler_params=pltpu.CompilerParams(
            dimension_semantics=("parallel",)),
        in_specs=[
            pl.BlockSpec((TC_BLOCK, 2 * H), lambda i: (i, 0)),
            pl.BlockSpec((TC_BLOCK, 2 * H), lambda i: (i, 0)),
            pl.BlockSpec((TC_BLOCK, 2 * H), lambda i: (i, 0)),
            pl.BlockSpec((200, TC_BLOCK), lambda i: (0, i)),
            pl.BlockSpec((64, TC_BLOCK), lambda i: (0, i)),
            pl.BlockSpec((TC_BLOCK, 128), lambda i: (i, 0)),
            pl.BlockSpec((6, 1, TC_BLOCK), lambda i: (0, 0, i)),
            pl.BlockSpec((64, 200), lambda i: (0, 0)),
            pl.BlockSpec((64, 64), lambda i: (0, 0)),
            pl.BlockSpec((64, 128), lambda i: (0, 0)),
            pl.BlockSpec((64, 3), lambda i: (0, 0)),
            pl.BlockSpec((6, H), lambda i: (0, 0)),
            pl.BlockSpec((2, H), lambda i: (0, 0)),
            pl.BlockSpec((18, H), lambda i: (0, 0)),
        ],
        out_specs=pl.BlockSpec((9 * H, TC_BLOCK), lambda i: (0, i)),
        out_shape=jax.ShapeDtypeStruct((9 * H, B), jnp.float32),
    )(gath0, gath1, gath2, art_t, mom_t, features, ids,
      wuf, wml, wsf, biases_t, age_emb, gender_emb, genre_emb)


def kernel(user_articles, user_moments, user_id, user_age, user_gender,
           music_features, music_singer, music_genre, music_id,
           W_uf, b_uf, W_ml, b_ml, W_sf, b_sf,
           UserEmb, AgeEmb, GenderEmb, SingerEmb, GenreEmb, MusicEmb):
    i32 = jnp.int32

    user_p = _pack_table(UserEmb.T, N_USER)
    singer_p = _pack_table(SingerEmb.T, N_SINGER)
    music_p = _pack_table(MusicEmb.T, N_MUSIC)

    def packed_row_parity(r):
        r = r.astype(i32)
        row = (r // STRIPE) * (STRIPE // 2) + (r % (STRIPE // 2))
        parity = (r // (STRIPE // 2)) % 2
        return row, parity

    row_u, par_u = packed_row_parity(user_id)
    row_s, par_s = packed_row_parity(music_singer.reshape(B))
    row_m, par_m = packed_row_parity(music_id.reshape(B))

    gath0, gath1, gath2 = _sc_gather(user_p, singer_p, music_p,
                                     row_u, row_s, row_m)

    ids = jnp.stack([user_age.astype(i32),
                     user_gender.astype(i32),
                     music_genre.reshape(B).astype(i32),
                     par_u, par_s, par_m], axis=0).reshape(6, 1, B)
    biases_t = jnp.stack([b_uf, b_ml, b_sf], axis=1)
    out_t = _tc_assemble(gath0, gath1, gath2, user_articles.T,
                         user_moments.T, music_features.reshape(B, 128),
                         ids, W_uf, W_ml, W_sf, biases_t,
                         AgeEmb, GenderEmb, GenreEmb)
    return out_t.T


# speedup vs baseline: 1.3183x; 1.0014x over previous
import functools

import jax
import jax.numpy as jnp
from jax import lax
from jax.experimental import pallas as pl
from jax.experimental.pallas import tpu as pltpu
from jax.experimental.pallas import tpu_sc as plsc

B = 16384
H = 64
NC = 2
NS = 16
NW = NC * NS
B_PER_W = B // NW

TC_BLOCK = 1024
GRID = B // TC_BLOCK

STRIPE = 8192
N_USER = 190662
N_MUSIC = 42800
N_SINGER = 417


def _packed_rows(m):
    return ((m + STRIPE - 1) // STRIPE) * (STRIPE // 2)


def _pack_kernel(tab_t_ref, out_ref):
    q = STRIPE // 4
    s0 = tab_t_ref[:, 0:q].T
    s1 = tab_t_ref[:, q:2 * q].T
    s2 = tab_t_ref[:, 2 * q:3 * q].T
    s3 = tab_t_ref[:, 3 * q:4 * q].T
    left = jnp.concatenate([s0, s1], axis=0)
    right = jnp.concatenate([s2, s3], axis=0)
    out_ref[...] = jnp.concatenate([left, right], axis=1)


def _pack_table(table_t, m):
    grid = (m + STRIPE - 1) // STRIPE
    return pl.pallas_call(
        _pack_kernel,
        grid=(grid,),
        compiler_params=pltpu.CompilerParams(
            dimension_semantics=("parallel",)),
        in_specs=[pl.BlockSpec((H, STRIPE), lambda i: (0, i))],
        out_specs=pl.BlockSpec((STRIPE // 2, 2 * H), lambda i: (i, 0)),
        out_shape=jax.ShapeDtypeStruct((grid * (STRIPE // 2), 2 * H),
                                       jnp.float32),
    )(table_t)


CHUNK = B_PER_W // 2


def _chunked_gather(jobs, idx_bufs, rows_bufs, gsem, wsems, base):
    chunks = []
    for j, (table, idx_hbm, out_hbm) in enumerate(jobs):
        pltpu.sync_copy(idx_hbm.at[pl.ds(base, B_PER_W)], idx_bufs[j])
        for c in range(B_PER_W // CHUNK):
            chunks.append((j, c))
    wbs = [None] * len(chunks)
    for k, (j, c) in enumerate(chunks):
        buf = rows_bufs[k % 2]
        if k >= 2 and wbs[k - 2] is not None:
            wbs[k - 2].wait()
        table, _, out_hbm = jobs[j]
        pltpu.async_copy(table.at[idx_bufs[j].at[pl.ds(c * CHUNK, CHUNK)]],
                         buf, gsem).wait()
        wbs[k] = pltpu.async_copy(
            buf, out_hbm.at[pl.ds(base + c * CHUNK, CHUNK)], wsems[k % 2])
    for k in range(max(0, len(chunks) - 2), len(chunks)):
        wbs[k].wait()


def _sc_gather2_kernel(tab_a, tab_b, idx_a, idx_b, out_a, out_b,
                       idx_v0, idx_v1, rows_v0, rows_v1, gsem, wsem0, wsem1):
    wid = lax.axis_index("s") * NC + lax.axis_index("c")
    base = wid * B_PER_W
    _chunked_gather([(tab_a, idx_a, out_a), (tab_b, idx_b, out_b)],
                    [idx_v0, idx_v1], [rows_v0, rows_v1],
                    gsem, [wsem0, wsem1], base)


def _sc_gather1_kernel(table, idx, out, idx_v, rows_v0, rows_v1,
                       gsem, wsem0, wsem1):
    wid = lax.axis_index("s") * NC + lax.axis_index("c")
    base = wid * B_PER_W
    _chunked_gather([(table, idx, out)],
                    [idx_v], [rows_v0, rows_v1],
                    gsem, [wsem0, wsem1], base)


_SC_MESH_KW = dict(
    compiler_params=pltpu.CompilerParams(use_tc_tiling_on_sc=True),
)


def _sc_gather_pair(tab_a, tab_b, idx_a, idx_b):
    mesh = plsc.VectorSubcoreMesh(core_axis_name="c", subcore_axis_name="s")
    out = jax.ShapeDtypeStruct((B, 2 * H), jnp.float32)
    k = pl.kernel(
        _sc_gather2_kernel,
        out_type=(out, out),
        mesh=mesh,
        scratch_types=[
            pltpu.VMEM((B_PER_W,), jnp.int32),
            pltpu.VMEM((B_PER_W,), jnp.int32),
            pltpu.VMEM((CHUNK, 2 * H), jnp.float32),
            pltpu.VMEM((CHUNK, 2 * H), jnp.float32),
            pltpu.SemaphoreType.DMA,
            pltpu.SemaphoreType.DMA,
            pltpu.SemaphoreType.DMA,
        ],
        **_SC_MESH_KW,
    )
    return k(tab_a, tab_b, idx_a, idx_b)


def _sc_gather_one(table, idx):
    mesh = plsc.VectorSubcoreMesh(core_axis_name="c", subcore_axis_name="s")
    out = jax.ShapeDtypeStruct((B, 2 * H), jnp.float32)
    k = pl.kernel(
        _sc_gather1_kernel,
        out_type=out,
        mesh=mesh,
        scratch_types=[
            pltpu.VMEM((B_PER_W,), jnp.int32),
            pltpu.VMEM((CHUNK, 2 * H), jnp.float32),
            pltpu.VMEM((CHUNK, 2 * H), jnp.float32),
            pltpu.SemaphoreType.DMA,
            pltpu.SemaphoreType.DMA,
            pltpu.SemaphoreType.DMA,
        ],
        **_SC_MESH_KW,
    )
    return k(table, idx)


def _tc_kernel(gath0_ref, gath1_ref, gath2_ref, art_t_ref, mom_t_ref,
               feat_ref, ids_ref, wuf_ref, wml_ref, wsf_ref, bias_t_ref,
               age_emb_ref, gender_emb_ref, genre_emb_ref, out_ref):
    f32 = jnp.float32
    hi = jax.lax.Precision.DEFAULT

    def onehot_lookup_t(f, n, table_ref):
        ids_row = ids_ref[f]
        iota = lax.broadcasted_iota(jnp.int32, (n, TC_BLOCK), 0)
        oh = (iota == ids_row).astype(f32)
        return lax.dot_general(table_ref[...], oh,
                               (((0,), (0,)), ((), ())), precision=hi)

    def half_select_t(g_ref, par_row):
        g_t = g_ref[...].T
        return jnp.where(par_row == 1, g_t[H:2 * H, :], g_t[0:H, :])

    age_t = onehot_lookup_t(0, 6, age_emb_ref)
    gender_t = onehot_lookup_t(1, 2, gender_emb_ref)
    genre_t = onehot_lookup_t(2, 18, genre_emb_ref)

    uemb_t = half_select_t(gath0_ref, ids_ref[3])
    semb_t = half_select_t(gath1_ref, ids_ref[4])
    memb_t = half_select_t(gath2_ref, ids_ref[5])

    art_t = lax.dot_general(wuf_ref[...], art_t_ref[...],
                            (((1,), (0,)), ((), ())),
                            precision=hi) + bias_t_ref[:, 0:1]
    mom_t = lax.dot_general(wml_ref[...], mom_t_ref[...],
                            (((1,), (0,)), ((), ())),
                            precision=hi) + bias_t_ref[:, 1:2]
    feat_t = lax.dot_general(wsf_ref[...], feat_ref[...],
                             (((1,), (1,)), ((), ())),
                             precision=hi) + bias_t_ref[:, 2:3]

    out_ref[...] = jnp.concatenate(
        [uemb_t, age_t, gender_t, art_t, mom_t, feat_t,
         semb_t, genre_t, memb_t], axis=0)


def _tc_assemble(gath0, gath1, gath2, art_t, mom_t, features, ids,
                 wuf, wml, wsf, biases_t, age_emb, gender_emb, genre_emb):
    return pl.pallas_call(
        _tc_kernel,
        grid=(GRID,),
        compiler_params=pltpu.CompilerParams(
            dimension_semantics=("parallel",)),
        in_specs=[
            pl.BlockSpec((TC_BLOCK, 2 * H), lambda i: (i, 0)),
            pl.BlockSpec((TC_BLOCK, 2 * H), lambda i: (i, 0)),
            pl.BlockSpec((TC_BLOCK, 2 * H), lambda i: (i, 0)),
            pl.BlockSpec((200, TC_BLOCK), lambda i: (0, i)),
            pl.BlockSpec((64, TC_BLOCK), lambda i: (0, i)),
            pl.BlockSpec((TC_BLOCK, 128), lambda i: (i, 0)),
            pl.BlockSpec((6, 1, TC_BLOCK), lambda i: (0, 0, i)),
            pl.BlockSpec((64, 200), lambda i: (0, 0)),
            pl.BlockSpec((64, 64), lambda i: (0, 0)),
            pl.BlockSpec((64, 128), lambda i: (0, 0)),
            pl.BlockSpec((64, 3), lambda i: (0, 0)),
            pl.BlockSpec((6, H), lambda i: (0, 0)),
            pl.BlockSpec((2, H), lambda i: (0, 0)),
            pl.BlockSpec((18, H), lambda i: (0, 0)),
        ],
        out_specs=pl.BlockSpec((9 * H, TC_BLOCK), lambda i: (0, i)),
        out_shape=jax.ShapeDtypeStruct((9 * H, B), jnp.float32),
    )(gath0, gath1, gath2, art_t, mom_t, features, ids,
      wuf, wml, wsf, biases_t, age_emb, gender_emb, genre_emb)


def kernel(user_articles, user_moments, user_id, user_age, user_gender,
           music_features, music_singer, music_genre, music_id,
           W_uf, b_uf, W_ml, b_ml, W_sf, b_sf,
           UserEmb, AgeEmb, GenderEmb, SingerEmb, GenreEmb, MusicEmb):
    i32 = jnp.int32

    singer_p = _pack_table(SingerEmb.T, N_SINGER)
    music_p = _pack_table(MusicEmb.T, N_MUSIC)
    user_p = _pack_table(UserEmb.T, N_USER)

    def packed_row_parity(r):
        r = r.astype(i32)
        row = (r // STRIPE) * (STRIPE // 2) + (r % (STRIPE // 2))
        parity = (r // (STRIPE // 2)) % 2
        return row, parity

    row_u, par_u = packed_row_parity(user_id)
    row_s, par_s = packed_row_parity(music_singer.reshape(B))
    row_m, par_m = packed_row_parity(music_id.reshape(B))

    gath1, gath2 = _sc_gather_pair(singer_p, music_p, row_s, row_m)
    gath0 = _sc_gather_one(user_p, row_u)

    ids = jnp.stack([user_age.astype(i32),
                     user_gender.astype(i32),
                     music_genre.reshape(B).astype(i32),
                     par_u, par_s, par_m], axis=0).reshape(6, 1, B)
    biases_t = jnp.stack([b_uf, b_ml, b_sf], axis=1)
    out_t = _tc_assemble(gath0, gath1, gath2, user_articles.T,
                         user_moments.T, music_features.reshape(B, 128),
                         ids, W_uf, W_ml, W_sf, biases_t,
                         AgeEmb, GenderEmb, GenreEmb)
    return out_t.T


# speedup vs baseline: 1.3498x; 1.0239x over previous
import functools

import jax
import jax.numpy as jnp
from jax import lax
from jax.experimental import pallas as pl
from jax.experimental.pallas import tpu as pltpu
from jax.experimental.pallas import tpu_sc as plsc

B = 16384
H = 64
NC = 2
NS = 16
NW = NC * NS
B_PER_W = B // NW

TC_BLOCK = 1024
GRID = B // TC_BLOCK

STRIPE = 16384
N_USER = 190662
N_MUSIC = 42800
N_SINGER = 417


def _packed_rows(m):
    return ((m + STRIPE - 1) // STRIPE) * (STRIPE // 2)


def _pack_kernel(tab_t_ref, out_ref):
    q = STRIPE // 4
    s0 = tab_t_ref[:, 0:q].T
    s1 = tab_t_ref[:, q:2 * q].T
    s2 = tab_t_ref[:, 2 * q:3 * q].T
    s3 = tab_t_ref[:, 3 * q:4 * q].T
    left = jnp.concatenate([s0, s1], axis=0)
    right = jnp.concatenate([s2, s3], axis=0)
    out_ref[...] = jnp.concatenate([left, right], axis=1)


def _pack_table(table_t, m):
    grid = (m + STRIPE - 1) // STRIPE
    return pl.pallas_call(
        _pack_kernel,
        grid=(grid,),
        compiler_params=pltpu.CompilerParams(
            dimension_semantics=("parallel",)),
        in_specs=[pl.BlockSpec((H, STRIPE), lambda i: (0, i))],
        out_specs=pl.BlockSpec((STRIPE // 2, 2 * H), lambda i: (i, 0)),
        out_shape=jax.ShapeDtypeStruct((grid * (STRIPE // 2), 2 * H),
                                       jnp.float32),
    )(table_t)


CHUNK = B_PER_W // 2


def _chunked_gather(jobs, idx_bufs, rows_bufs, gsem, wsems, base):
    chunks = []
    for j, (table, idx_hbm, out_hbm) in enumerate(jobs):
        pltpu.sync_copy(idx_hbm.at[pl.ds(base, B_PER_W)], idx_bufs[j])
        for c in range(B_PER_W // CHUNK):
            chunks.append((j, c))
    wbs = [None] * len(chunks)
    for k, (j, c) in enumerate(chunks):
        buf = rows_bufs[k % 2]
        if k >= 2 and wbs[k - 2] is not None:
            wbs[k - 2].wait()
        table, _, out_hbm = jobs[j]
        pltpu.async_copy(table.at[idx_bufs[j].at[pl.ds(c * CHUNK, CHUNK)]],
                         buf, gsem).wait()
        wbs[k] = pltpu.async_copy(
            buf, out_hbm.at[pl.ds(base + c * CHUNK, CHUNK)], wsems[k % 2])
    for k in range(max(0, len(chunks) - 2), len(chunks)):
        wbs[k].wait()


def _sc_gather2_kernel(tab_a, tab_b, idx_a, idx_b, out_a, out_b,
                       idx_v0, idx_v1, rows_v0, rows_v1, gsem, wsem0, wsem1):
    wid = lax.axis_index("s") * NC + lax.axis_index("c")
    base = wid * B_PER_W
    _chunked_gather([(tab_a, idx_a, out_a), (tab_b, idx_b, out_b)],
                    [idx_v0, idx_v1], [rows_v0, rows_v1],
                    gsem, [wsem0, wsem1], base)


def _sc_gather1_kernel(table, idx, out, idx_v, rows_v0, rows_v1,
                       gsem, wsem0, wsem1):
    wid = lax.axis_index("s") * NC + lax.axis_index("c")
    base = wid * B_PER_W
    _chunked_gather([(table, idx, out)],
                    [idx_v], [rows_v0, rows_v1],
                    gsem, [wsem0, wsem1], base)


_SC_MESH_KW = dict(
    compiler_params=pltpu.CompilerParams(use_tc_tiling_on_sc=True),
)


def _sc_gather_pair(tab_a, tab_b, idx_a, idx_b):
    mesh = plsc.VectorSubcoreMesh(core_axis_name="c", subcore_axis_name="s")
    out = jax.ShapeDtypeStruct((B, 2 * H), jnp.float32)
    k = pl.kernel(
        _sc_gather2_kernel,
        out_type=(out, out),
        mesh=mesh,
        scratch_types=[
            pltpu.VMEM((B_PER_W,), jnp.int32),
            pltpu.VMEM((B_PER_W,), jnp.int32),
            pltpu.VMEM((CHUNK, 2 * H), jnp.float32),
            pltpu.VMEM((CHUNK, 2 * H), jnp.float32),
            pltpu.SemaphoreType.DMA,
            pltpu.SemaphoreType.DMA,
            pltpu.SemaphoreType.DMA,
        ],
        **_SC_MESH_KW,
    )
    return k(tab_a, tab_b, idx_a, idx_b)


def _sc_gather_one(table, idx):
    mesh = plsc.VectorSubcoreMesh(core_axis_name="c", subcore_axis_name="s")
    out = jax.ShapeDtypeStruct((B, 2 * H), jnp.float32)
    k = pl.kernel(
        _sc_gather1_kernel,
        out_type=out,
        mesh=mesh,
        scratch_types=[
            pltpu.VMEM((B_PER_W,), jnp.int32),
            pltpu.VMEM((CHUNK, 2 * H), jnp.float32),
            pltpu.VMEM((CHUNK, 2 * H), jnp.float32),
            pltpu.SemaphoreType.DMA,
            pltpu.SemaphoreType.DMA,
            pltpu.SemaphoreType.DMA,
        ],
        **_SC_MESH_KW,
    )
    return k(table, idx)


def _tc_kernel(gath0_ref, gath1_ref, gath2_ref, art_t_ref, mom_t_ref,
               feat_ref, ids_ref, wuf_ref, wml_ref, wsf_ref, bias_t_ref,
               age_emb_ref, gender_emb_ref, genre_emb_ref, out_ref):
    f32 = jnp.float32
    hi = jax.lax.Precision.DEFAULT

    def onehot_lookup_t(f, n, table_ref):
        ids_row = ids_ref[f]
        iota = lax.broadcasted_iota(jnp.int32, (n, TC_BLOCK), 0)
        oh = (iota == ids_row).astype(f32)
        return lax.dot_general(table_ref[...], oh,
                               (((0,), (0,)), ((), ())), precision=hi)

    def half_select_t(g_ref, par_row):
        g_t = g_ref[...].T
        return jnp.where(par_row == 1, g_t[H:2 * H, :], g_t[0:H, :])

    age_t = onehot_lookup_t(0, 6, age_emb_ref)
    gender_t = onehot_lookup_t(1, 2, gender_emb_ref)
    genre_t = onehot_lookup_t(2, 18, genre_emb_ref)

    uemb_t = half_select_t(gath0_ref, ids_ref[3])
    semb_t = half_select_t(gath1_ref, ids_ref[4])
    memb_t = half_select_t(gath2_ref, ids_ref[5])

    art_t = lax.dot_general(wuf_ref[...], art_t_ref[...],
                            (((1,), (0,)), ((), ())),
                            precision=hi) + bias_t_ref[:, 0:1]
    mom_t = lax.dot_general(wml_ref[...], mom_t_ref[...],
                            (((1,), (0,)), ((), ())),
                            precision=hi) + bias_t_ref[:, 1:2]
    feat_t = lax.dot_general(wsf_ref[...], feat_ref[...],
                             (((1,), (1,)), ((), ())),
                             precision=hi) + bias_t_ref[:, 2:3]

    out_ref[...] = jnp.concatenate(
        [uemb_t, age_t, gender_t, art_t, mom_t, feat_t,
         semb_t, genre_t, memb_t], axis=0)


def _tc_assemble(gath0, gath1, gath2, art_t, mom_t, features, ids,
                 wuf, wml, wsf, biases_t, age_emb, gender_emb, genre_emb):
    return pl.pallas_call(
        _tc_kernel,
        grid=(GRID,),
        compiler_params=pltpu.CompilerParams(
            dimension_semantics=("parallel",)),
        in_specs=[
            pl.BlockSpec((TC_BLOCK, 2 * H), lambda i: (i, 0)),
            pl.BlockSpec((TC_BLOCK, 2 * H), lambda i: (i, 0)),
            pl.BlockSpec((TC_BLOCK, 2 * H), lambda i: (i, 0)),
            pl.BlockSpec((200, TC_BLOCK), lambda i: (0, i)),
            pl.BlockSpec((64, TC_BLOCK), lambda i: (0, i)),
            pl.BlockSpec((TC_BLOCK, 128), lambda i: (i, 0)),
            pl.BlockSpec((6, 1, TC_BLOCK), lambda i: (0, 0, i)),
            pl.BlockSpec((64, 200), lambda i: (0, 0)),
            pl.BlockSpec((64, 64), lambda i: (0, 0)),
            pl.BlockSpec((64, 128), lambda i: (0, 0)),
            pl.BlockSpec((64, 3), lambda i: (0, 0)),
            pl.BlockSpec((6, H), lambda i: (0, 0)),
            pl.BlockSpec((2, H), lambda i: (0, 0)),
            pl.BlockSpec((18, H), lambda i: (0, 0)),
        ],
        out_specs=pl.BlockSpec((9 * H, TC_BLOCK), lambda i: (0, i)),
        out_shape=jax.ShapeDtypeStruct((9 * H, B), jnp.float32),
    )(gath0, gath1, gath2, art_t, mom_t, features, ids,
      wuf, wml, wsf, biases_t, age_emb, gender_emb, genre_emb)


def kernel(user_articles, user_moments, user_id, user_age, user_gender,
           music_features, music_singer, music_genre, music_id,
           W_uf, b_uf, W_ml, b_ml, W_sf, b_sf,
           UserEmb, AgeEmb, GenderEmb, SingerEmb, GenreEmb, MusicEmb):
    i32 = jnp.int32

    singer_p = _pack_table(SingerEmb.T, N_SINGER)
    music_p = _pack_table(MusicEmb.T, N_MUSIC)
    user_p = _pack_table(UserEmb.T, N_USER)

    def packed_row_parity(r):
        r = r.astype(i32)
        row = (r // STRIPE) * (STRIPE // 2) + (r % (STRIPE // 2))
        parity = (r // (STRIPE // 2)) % 2
        return row, parity

    row_u, par_u = packed_row_parity(user_id)
    row_s, par_s = packed_row_parity(music_singer.reshape(B))
    row_m, par_m = packed_row_parity(music_id.reshape(B))

    gath1, gath2 = _sc_gather_pair(singer_p, music_p, row_s, row_m)
    gath0 = _sc_gather_one(user_p, row_u)

    ids = jnp.stack([user_age.astype(i32),
                     user_gender.astype(i32),
                     music_genre.reshape(B).astype(i32),
                     par_u, par_s, par_m], axis=0).reshape(6, 1, B)
    biases_t = jnp.stack([b_uf, b_ml, b_sf], axis=1)
    out_t = _tc_assemble(gath0, gath1, gath2, user_articles.T,
                         user_moments.T, music_features.reshape(B, 128),
                         ids, W_uf, W_ml, W_sf, biases_t,
                         AgeEmb, GenderEmb, GenreEmb)
    return out_t.T
